# baseline (speedup 1.0000x reference)
import functools

import jax
import jax.numpy as jnp
from jax import lax
from jax.experimental import pallas as pl
from jax.experimental.pallas import tpu as pltpu
from jax.experimental.pallas import tpu_sc as plsc

N = 10000
E = 320000
NC, NS, L = 2, 16, 16
NW = NC * NS
EPT = E // NW
C = 112
NCHUNK = 90
EPT_P = NCHUNK * C
NP = 10240
RPT = NP // NS
RC = 80
NRC = RPT // RC


def _vreg_gather(v, idx):
    return lax.gather(
        v, idx[:, None],
        dimension_numbers=lax.GatherDimensionNumbers(
            offset_dims=(), collapsed_slice_dims=(0,), start_index_map=(0,)),
        slice_sizes=(1,),
        mode=lax.GatherScatterMode.PROMISE_IN_BOUNDS)


def _spmm_body(D, dst_h, src_h, w_h, sup_h, out_h,
               acc_sh, dst_v, src_v, w_v,
               gbuf0, gbuf1, sbuf0, sbuf1,
               gsem0, gsem1, ssem0, ssem1):
    c = lax.axis_index("c")
    s = lax.axis_index("s")
    wid = c * NS + s
    rbase = s * RPT
    zero16 = jnp.zeros((L,), jnp.float32)
    zi16 = jnp.zeros((L,), jnp.int32)

    @pl.loop(0, RC)
    def _zero_row(i):
        for j in range(D // L):
            gbuf0[i, pl.ds(j * L, L)] = zero16

    for r in range(NRC):
        pltpu.sync_copy(gbuf0.at[pl.ds(0, RC)],
                        acc_sh.at[pl.ds(rbase + r * RC, RC)])

    pltpu.sync_copy(dst_h.at[wid], dst_v)
    pltpu.sync_copy(src_h.at[wid], src_v)
    pltpu.sync_copy(w_h.at[wid], w_v)

    pltpu.async_copy(sup_h.at[src_v.at[0]], gbuf0, gsem0)
    pltpu.async_copy(sup_h.at[src_v.at[1]], gbuf1, gsem1)
    plsc.subcore_barrier()

    def process(j, gbuf, sbuf, gsem, ssem):
        pltpu.make_async_copy(sup_h.at[src_v.at[j]], gbuf, gsem).wait()

        @pl.loop(0, C // L)
        def _scale(g):
            w16 = w_v[j, pl.ds(g * L, L)]
            for ii in range(L):
                wb = _vreg_gather(w16, jnp.full((L,), ii, jnp.int32))
                i = g * L + ii
                for d in range(D // L):
                    sl = pl.ds(d * L, L)
                    sbuf[i, sl] = gbuf[i, sl] * wb

        @pl.when(j + 2 < NCHUNK)
        def _():
            pltpu.async_copy(sup_h.at[src_v.at[j + 2]], gbuf, gsem)

        @pl.when(j >= 2)
        def _():
            pltpu.make_async_copy(sbuf, acc_sh.at[dst_v.at[j - 2]],
                                  ssem).wait()
        pltpu.async_copy(sbuf, acc_sh.at[dst_v.at[j]], ssem, add=True)

    @pl.loop(0, NCHUNK - (NCHUNK % 2), step=2)
    def _pair(k):
        process(k, gbuf0, sbuf0, gsem0, ssem0)
        process(k + 1, gbuf1, sbuf1, gsem1, ssem1)

    if NCHUNK % 2:
        process(NCHUNK - 1, gbuf0, sbuf0, gsem0, ssem0)
        pltpu.make_async_copy(sbuf1, acc_sh.at[dst_v.at[NCHUNK - 2]],
                              ssem1).wait()
        pltpu.make_async_copy(sbuf0, acc_sh.at[dst_v.at[NCHUNK - 1]],
                              ssem0).wait()
    else:
        pltpu.make_async_copy(sbuf0, acc_sh.at[dst_v.at[NCHUNK - 2]],
                              ssem0).wait()
        pltpu.make_async_copy(sbuf1, acc_sh.at[dst_v.at[NCHUNK - 1]],
                              ssem1).wait()
    plsc.subcore_barrier()

    for r in range(NRC):
        pltpu.sync_copy(acc_sh.at[pl.ds(rbase + r * RC, RC)],
                        gbuf0.at[pl.ds(0, RC)])
        pltpu.sync_copy(gbuf0.at[pl.ds(0, RC)],
                        out_h.at[c, pl.ds(rbase + r * RC, RC)])


@functools.cache
def _make_spmm(D):
    mesh = plsc.VectorSubcoreMesh(core_axis_name="c", subcore_axis_name="s",
                                  num_cores=NC, num_subcores=NS)
    return pl.kernel(
        functools.partial(_spmm_body, D),
        out_type=jax.ShapeDtypeStruct((NC, NP, D), jnp.float32),
        mesh=mesh,
        scratch_types=[
            pltpu.VMEM_SHARED((NP, D), jnp.float32),
            pltpu.VMEM((NCHUNK, C), jnp.int32),
            pltpu.VMEM((NCHUNK, C), jnp.int32),
            pltpu.VMEM((NCHUNK, C), jnp.float32),
            pltpu.VMEM((C, D), jnp.float32),
            pltpu.VMEM((C, D), jnp.float32),
            pltpu.VMEM((C, D), jnp.float32),
            pltpu.VMEM((C, D), jnp.float32),
            pltpu.SemaphoreType.DMA,
            pltpu.SemaphoreType.DMA,
            pltpu.SemaphoreType.DMA,
            pltpu.SemaphoreType.DMA,
        ],
        compiler_params=pltpu.CompilerParams(needs_layout_passes=False,
                                             use_tc_tiling_on_sc=False),
    )


_BLK = 1000


def _mm_body(x_ref, w_ref, o_ref):
    o_ref[...] = jnp.dot(x_ref[...], w_ref[...],
                         preferred_element_type=jnp.float32)


def _matmul(x, W):
    M, K = x.shape
    Dout = W.shape[1]
    return pl.pallas_call(
        _mm_body,
        grid=(M // _BLK,),
        in_specs=[pl.BlockSpec((_BLK, K), lambda i: (i, 0)),
                  pl.BlockSpec((K, Dout), lambda i: (0, 0))],
        out_specs=pl.BlockSpec((_BLK, Dout), lambda i: (i, 0)),
        out_shape=jax.ShapeDtypeStruct((M, Dout), jnp.float32),
    )(x, W)


def _cmm_body(p_ref, b_ref, w_ref, act_ref, sup_ref):
    act = jnp.maximum(p_ref[0] + p_ref[1] + b_ref[...], 0.0)
    act_ref[...] = act
    sup_ref[...] = jnp.dot(act, w_ref[...], preferred_element_type=jnp.float32)


def _combine_mm(P, b, W):
    D = P.shape[2]
    Dout = W.shape[1]
    return pl.pallas_call(
        _cmm_body,
        grid=(N // _BLK,),
        in_specs=[pl.BlockSpec((2, _BLK, D), lambda i: (0, i, 0)),
                  pl.BlockSpec((1, D), lambda i: (0, 0)),
                  pl.BlockSpec((D, Dout), lambda i: (0, 0))],
        out_specs=[pl.BlockSpec((_BLK, D), lambda i: (i, 0)),
                   pl.BlockSpec((_BLK, Dout), lambda i: (i, 0))],
        out_shape=[jax.ShapeDtypeStruct((N, D), jnp.float32),
                   jax.ShapeDtypeStruct((N, Dout), jnp.float32)],
    )(P, b.reshape(1, D), W)


def _c2_body(pa_ref, pb_ref, b_ref, act_ref):
    act = jnp.concatenate([pa_ref[0] + pa_ref[1], pb_ref[0] + pb_ref[1]],
                          axis=1)
    act_ref[...] = jnp.maximum(act + b_ref[...], 0.0)


def _combine2(Pa, Pb, b):
    Dh = Pa.shape[2]
    D = 2 * Dh
    return pl.pallas_call(
        _c2_body,
        grid=(N // _BLK,),
        in_specs=[pl.BlockSpec((2, _BLK, Dh), lambda i: (0, i, 0)),
                  pl.BlockSpec((2, _BLK, Dh), lambda i: (0, i, 0)),
                  pl.BlockSpec((1, D), lambda i: (0, 0))],
        out_specs=pl.BlockSpec((_BLK, D), lambda i: (i, 0)),
        out_shape=jax.ShapeDtypeStruct((N, D), jnp.float32),
    )(Pa, Pb, b.reshape(1, D))


def _branch(feat, ew, ei, Ws, bs):
    pad = ((0, 0), (0, EPT_P - EPT))
    dst = jnp.pad(ei[0].astype(jnp.int32).reshape(NW, EPT),
                  pad).reshape(NW, NCHUNK, C)
    src = jnp.pad(ei[1].astype(jnp.int32).reshape(NW, EPT),
                  pad).reshape(NW, NCHUNK, C)
    ew = jnp.pad(ew.reshape(NW, EPT), pad).reshape(NW, NCHUNK, C)
    s0 = _matmul(feat, Ws[0])
    P = _make_spmm(Ws[0].shape[1])(dst, src, ew, s0)
    _, s1 = _combine_mm(P, bs[0], Ws[1])
    P = _make_spmm(Ws[1].shape[1])(dst, src, ew, s1)
    p0, s2 = _combine_mm(P, bs[1], Ws[2])
    P = _make_spmm(Ws[2].shape[1])(dst, src, ew, s2)
    _, s3 = _combine_mm(P, bs[2], Ws[3])
    Dh = Ws[3].shape[1] // 2
    Pa = _make_spmm(Dh)(dst, src, ew, s3[:, :Dh])
    Pb = _make_spmm(Dh)(dst, src, ew, s3[:, Dh:])
    h1 = _combine2(Pa, Pb, bs[3])
    return p0, h1


def kernel(feature1, edge_weight1, feature2, edge_weight2, params,
           edge_index1, edge_index2):
    p0, h1 = _branch(feature1, edge_weight1, edge_index1,
                     params["Ws1"], params["bs1"])
    p3, h4 = _branch(feature2, edge_weight2, edge_index2,
                     params["Ws2"], params["bs2"])
    return (p0, h1, p3, h4)


# speedup vs baseline: 1.0018x; 1.0018x over previous
import functools

import jax
import jax.numpy as jnp
from jax import lax
from jax.experimental import pallas as pl
from jax.experimental.pallas import tpu as pltpu
from jax.experimental.pallas import tpu_sc as plsc

N = 10000
E = 320000
NC, NS, L = 2, 16, 16
NW = NC * NS
EPT = E // NW
C = 64
NCHUNK = 157
EPT_P = NCHUNK * C
NP = 10240
RPT = NP // NS
RC = 64
NRC = RPT // RC


def _vreg_gather(v, idx):
    return lax.gather(
        v, idx[:, None],
        dimension_numbers=lax.GatherDimensionNumbers(
            offset_dims=(), collapsed_slice_dims=(0,), start_index_map=(0,)),
        slice_sizes=(1,),
        mode=lax.GatherScatterMode.PROMISE_IN_BOUNDS)


def _spmm_body(D, dst_h, src_h, w_h, sup_h, out_h,
               acc_sh, dst_v, src_v, w_v,
               gbuf0, gbuf1, sbuf0, sbuf1,
               gsem0, gsem1, ssem0, ssem1):
    c = lax.axis_index("c")
    s = lax.axis_index("s")
    wid = c * NS + s
    rbase = s * RPT
    zero16 = jnp.zeros((L,), jnp.float32)
    zi16 = jnp.zeros((L,), jnp.int32)

    @pl.loop(0, RC)
    def _zero_row(i):
        for j in range(D // L):
            gbuf0[i, pl.ds(j * L, L)] = zero16

    for r in range(NRC):
        pltpu.sync_copy(gbuf0.at[pl.ds(0, RC)],
                        acc_sh.at[pl.ds(rbase + r * RC, RC)])

    pltpu.sync_copy(dst_h.at[wid], dst_v)
    pltpu.sync_copy(src_h.at[wid], src_v)
    pltpu.sync_copy(w_h.at[wid], w_v)

    pltpu.async_copy(sup_h.at[src_v.at[0]], gbuf0, gsem0)
    pltpu.async_copy(sup_h.at[src_v.at[1]], gbuf1, gsem1)
    plsc.subcore_barrier()

    def process(j, gbuf, sbuf, gsem, ssem):
        pltpu.make_async_copy(sup_h.at[src_v.at[j]], gbuf, gsem).wait()

        @pl.loop(0, C // L)
        def _scale(g):
            w16 = w_v[j, pl.ds(g * L, L)]
            for ii in range(L):
                wb = _vreg_gather(w16, jnp.full((L,), ii, jnp.int32))
                i = g * L + ii
                for d in range(D // L):
                    sl = pl.ds(d * L, L)
                    sbuf[i, sl] = gbuf[i, sl] * wb

        @pl.when(j + 2 < NCHUNK)
        def _():
            pltpu.async_copy(sup_h.at[src_v.at[j + 2]], gbuf, gsem)

        @pl.when(j >= 2)
        def _():
            pltpu.make_async_copy(sbuf, acc_sh.at[dst_v.at[j - 2]],
                                  ssem).wait()
        pltpu.async_copy(sbuf, acc_sh.at[dst_v.at[j]], ssem, add=True)

    @pl.loop(0, NCHUNK - (NCHUNK % 2), step=2)
    def _pair(k):
        process(k, gbuf0, sbuf0, gsem0, ssem0)
        process(k + 1, gbuf1, sbuf1, gsem1, ssem1)

    if NCHUNK % 2:
        process(NCHUNK - 1, gbuf0, sbuf0, gsem0, ssem0)
        pltpu.make_async_copy(sbuf1, acc_sh.at[dst_v.at[NCHUNK - 2]],
                              ssem1).wait()
        pltpu.make_async_copy(sbuf0, acc_sh.at[dst_v.at[NCHUNK - 1]],
                              ssem0).wait()
    else:
        pltpu.make_async_copy(sbuf0, acc_sh.at[dst_v.at[NCHUNK - 2]],
                              ssem0).wait()
        pltpu.make_async_copy(sbuf1, acc_sh.at[dst_v.at[NCHUNK - 1]],
                              ssem1).wait()
    plsc.subcore_barrier()

    for r in range(NRC):
        pltpu.sync_copy(acc_sh.at[pl.ds(rbase + r * RC, RC)],
                        gbuf0.at[pl.ds(0, RC)])
        pltpu.sync_copy(gbuf0.at[pl.ds(0, RC)],
                        out_h.at[c, pl.ds(rbase + r * RC, RC)])


@functools.cache
def _make_spmm(D):
    mesh = plsc.VectorSubcoreMesh(core_axis_name="c", subcore_axis_name="s",
                                  num_cores=NC, num_subcores=NS)
    return pl.kernel(
        functools.partial(_spmm_body, D),
        out_type=jax.ShapeDtypeStruct((NC, NP, D), jnp.float32),
        mesh=mesh,
        scratch_types=[
            pltpu.VMEM_SHARED((NP, D), jnp.float32),
            pltpu.VMEM((NCHUNK, C), jnp.int32),
            pltpu.VMEM((NCHUNK, C), jnp.int32),
            pltpu.VMEM((NCHUNK, C), jnp.float32),
            pltpu.VMEM((C, D), jnp.float32),
            pltpu.VMEM((C, D), jnp.float32),
            pltpu.VMEM((C, D), jnp.float32),
            pltpu.VMEM((C, D), jnp.float32),
            pltpu.SemaphoreType.DMA,
            pltpu.SemaphoreType.DMA,
            pltpu.SemaphoreType.DMA,
            pltpu.SemaphoreType.DMA,
        ],
        compiler_params=pltpu.CompilerParams(needs_layout_passes=False,
                                             use_tc_tiling_on_sc=False),
    )


_BLK = 1000


def _mm_body(x_ref, w_ref, o_ref):
    o_ref[...] = jnp.dot(x_ref[...], w_ref[...],
                         preferred_element_type=jnp.float32)


def _matmul(x, W):
    M, K = x.shape
    Dout = W.shape[1]
    return pl.pallas_call(
        _mm_body,
        grid=(M // _BLK,),
        in_specs=[pl.BlockSpec((_BLK, K), lambda i: (i, 0)),
                  pl.BlockSpec((K, Dout), lambda i: (0, 0))],
        out_specs=pl.BlockSpec((_BLK, Dout), lambda i: (i, 0)),
        out_shape=jax.ShapeDtypeStruct((M, Dout), jnp.float32),
    )(x, W)


def _cmm_body(p_ref, b_ref, w_ref, act_ref, sup_ref):
    act = jnp.maximum(p_ref[0] + p_ref[1] + b_ref[...], 0.0)
    act_ref[...] = act
    sup_ref[...] = jnp.dot(act, w_ref[...], preferred_element_type=jnp.float32)


def _combine_mm(P, b, W):
    D = P.shape[2]
    Dout = W.shape[1]
    return pl.pallas_call(
        _cmm_body,
        grid=(N // _BLK,),
        in_specs=[pl.BlockSpec((2, _BLK, D), lambda i: (0, i, 0)),
                  pl.BlockSpec((1, D), lambda i: (0, 0)),
                  pl.BlockSpec((D, Dout), lambda i: (0, 0))],
        out_specs=[pl.BlockSpec((_BLK, D), lambda i: (i, 0)),
                   pl.BlockSpec((_BLK, Dout), lambda i: (i, 0))],
        out_shape=[jax.ShapeDtypeStruct((N, D), jnp.float32),
                   jax.ShapeDtypeStruct((N, Dout), jnp.float32)],
    )(P, b.reshape(1, D), W)


def _c2_body(pa_ref, pb_ref, b_ref, act_ref):
    act = jnp.concatenate([pa_ref[0] + pa_ref[1], pb_ref[0] + pb_ref[1]],
                          axis=1)
    act_ref[...] = jnp.maximum(act + b_ref[...], 0.0)


def _combine2(Pa, Pb, b):
    Dh = Pa.shape[2]
    D = 2 * Dh
    return pl.pallas_call(
        _c2_body,
        grid=(N // _BLK,),
        in_specs=[pl.BlockSpec((2, _BLK, Dh), lambda i: (0, i, 0)),
                  pl.BlockSpec((2, _BLK, Dh), lambda i: (0, i, 0)),
                  pl.BlockSpec((1, D), lambda i: (0, 0))],
        out_specs=pl.BlockSpec((_BLK, D), lambda i: (i, 0)),
        out_shape=jax.ShapeDtypeStruct((N, D), jnp.float32),
    )(Pa, Pb, b.reshape(1, D))


def _branch(feat, ew, ei, Ws, bs):
    pad = ((0, 0), (0, EPT_P - EPT))
    dst = jnp.pad(ei[0].astype(jnp.int32).reshape(NW, EPT),
                  pad).reshape(NW, NCHUNK, C)
    src = jnp.pad(ei[1].astype(jnp.int32).reshape(NW, EPT),
                  pad).reshape(NW, NCHUNK, C)
    ew = jnp.pad(ew.reshape(NW, EPT), pad).reshape(NW, NCHUNK, C)
    s0 = _matmul(feat, Ws[0])
    P = _make_spmm(Ws[0].shape[1])(dst, src, ew, s0)
    _, s1 = _combine_mm(P, bs[0], Ws[1])
    P = _make_spmm(Ws[1].shape[1])(dst, src, ew, s1)
    p0, s2 = _combine_mm(P, bs[1], Ws[2])
    P = _make_spmm(Ws[2].shape[1])(dst, src, ew, s2)
    _, s3 = _combine_mm(P, bs[2], Ws[3])
    Dh = Ws[3].shape[1] // 2
    Pa = _make_spmm(Dh)(dst, src, ew, s3[:, :Dh])
    Pb = _make_spmm(Dh)(dst, src, ew, s3[:, Dh:])
    h1 = _combine2(Pa, Pb, bs[3])
    return p0, h1


def kernel(feature1, edge_weight1, feature2, edge_weight2, params,
           edge_index1, edge_index2):
    p0, h1 = _branch(feature1, edge_weight1, edge_index1,
                     params["Ws1"], params["bs1"])
    p3, h4 = _branch(feature2, edge_weight2, edge_index2,
                     params["Ws2"], params["bs2"])
    return (p0, h1, p3, h4)


# speedup vs baseline: 1.3939x; 1.3914x over previous
import functools

import jax
import jax.numpy as jnp
from jax import lax
from jax.experimental import pallas as pl
from jax.experimental.pallas import tpu as pltpu
from jax.experimental.pallas import tpu_sc as plsc

N = 10000
E = 320000
NC, NS, L = 2, 16, 16
NW = NC * NS
EPT = E // NW
C = 80
NCHUNK = 125
EPT_P = NCHUNK * C
NP = 10240
RPT = NP // NS
RC = 80
NRC = RPT // RC


def _vreg_gather(v, idx):
    return lax.gather(
        v, idx[:, None],
        dimension_numbers=lax.GatherDimensionNumbers(
            offset_dims=(), collapsed_slice_dims=(0,), start_index_map=(0,)),
        slice_sizes=(1,),
        mode=lax.GatherScatterMode.PROMISE_IN_BOUNDS)


def _spmm_body(D, dst_h, src_h, w_h, sup_h, out_h,
               acc_sh, dst_v, src_v, w_v,
               gbuf0, gbuf1, sbuf0, sbuf1,
               gsem0, gsem1, ssem0, ssem1):
    c = lax.axis_index("c")
    s = lax.axis_index("s")
    wid = c * NS + s
    rbase = s * RPT
    zero16 = jnp.zeros((L,), jnp.float32)
    zi16 = jnp.zeros((L,), jnp.int32)

    @pl.loop(0, RC)
    def _zero_row(i):
        for j in range(D // L):
            gbuf0[i, pl.ds(j * L, L)] = zero16

    for r in range(NRC):
        pltpu.async_copy(gbuf0.at[pl.ds(0, RC)],
                         acc_sh.at[pl.ds(rbase + r * RC, RC)], gsem0)

    pltpu.async_copy(dst_h.at[wid], dst_v, gsem1)
    pltpu.async_copy(src_h.at[wid], src_v, gsem1)
    pltpu.async_copy(w_h.at[wid], w_v, gsem1)
    pltpu.make_async_copy(dst_h.at[wid], dst_v, gsem1).wait()
    pltpu.make_async_copy(src_h.at[wid], src_v, gsem1).wait()
    pltpu.make_async_copy(w_h.at[wid], w_v, gsem1).wait()
    for r in range(NRC):
        pltpu.make_async_copy(gbuf0.at[pl.ds(0, RC)],
                              acc_sh.at[pl.ds(rbase + r * RC, RC)],
                              gsem0).wait()

    pltpu.async_copy(sup_h.at[src_v.at[0]], gbuf0, gsem0)
    pltpu.async_copy(sup_h.at[src_v.at[1]], gbuf1, gsem1)
    plsc.subcore_barrier()

    def process(j, gbuf, sbuf, gsem, ssem, refill, waitprev):
        pltpu.make_async_copy(sup_h.at[src_v.at[j]], gbuf, gsem).wait()

        @pl.loop(0, C // L)
        def _scale(g):
            w16 = w_v[j, pl.ds(g * L, L)]
            for ii in range(L):
                wb = _vreg_gather(w16, jnp.full((L,), ii, jnp.int32))
                i = g * L + ii
                for d in range(D // L):
                    sl = pl.ds(d * L, L)
                    sbuf[i, sl] = gbuf[i, sl] * wb

        if refill:
            pltpu.async_copy(sup_h.at[src_v.at[j + 2]], gbuf, gsem)

        if waitprev:
            pltpu.make_async_copy(sbuf, acc_sh.at[dst_v.at[j - 2]],
                                  ssem).wait()
        pltpu.async_copy(sbuf, acc_sh.at[dst_v.at[j]], ssem, add=True)

    process(0, gbuf0, sbuf0, gsem0, ssem0, True, False)
    process(1, gbuf1, sbuf1, gsem1, ssem1, True, False)

    @pl.loop(2, NCHUNK - 3, step=2)
    def _pair(k):
        process(k, gbuf0, sbuf0, gsem0, ssem0, True, True)
        process(k + 1, gbuf1, sbuf1, gsem1, ssem1, True, True)

    process(NCHUNK - 3, gbuf0, sbuf0, gsem0, ssem0, True, True)
    process(NCHUNK - 2, gbuf1, sbuf1, gsem1, ssem1, False, True)
    process(NCHUNK - 1, gbuf0, sbuf0, gsem0, ssem0, False, True)
    pltpu.make_async_copy(sbuf1, acc_sh.at[dst_v.at[NCHUNK - 2]],
                          ssem1).wait()
    pltpu.make_async_copy(sbuf0, acc_sh.at[dst_v.at[NCHUNK - 1]],
                          ssem0).wait()
    plsc.subcore_barrier()

    for r in range(NRC):
        pltpu.async_copy(acc_sh.at[pl.ds(rbase + r * RC, RC)],
                         out_h.at[c, pl.ds(rbase + r * RC, RC)], gsem0)
    for r in range(NRC):
        pltpu.make_async_copy(acc_sh.at[pl.ds(rbase + r * RC, RC)],
                              out_h.at[c, pl.ds(rbase + r * RC, RC)],
                              gsem0).wait()


@functools.cache
def _make_spmm(D):
    mesh = plsc.VectorSubcoreMesh(core_axis_name="c", subcore_axis_name="s",
                                  num_cores=NC, num_subcores=NS)
    return pl.kernel(
        functools.partial(_spmm_body, D),
        out_type=jax.ShapeDtypeStruct((NC, NP, D), jnp.float32),
        mesh=mesh,
        scratch_types=[
            pltpu.VMEM_SHARED((NP, D), jnp.float32),
            pltpu.VMEM((NCHUNK, C), jnp.int32),
            pltpu.VMEM((NCHUNK, C), jnp.int32),
            pltpu.VMEM((NCHUNK, C), jnp.float32),
            pltpu.VMEM((C, D), jnp.float32),
            pltpu.VMEM((C, D), jnp.float32),
            pltpu.VMEM((C, D), jnp.float32),
            pltpu.VMEM((C, D), jnp.float32),
            pltpu.SemaphoreType.DMA,
            pltpu.SemaphoreType.DMA,
            pltpu.SemaphoreType.DMA,
            pltpu.SemaphoreType.DMA,
        ],
        compiler_params=pltpu.CompilerParams(needs_layout_passes=False,
                                             use_tc_tiling_on_sc=False),
    )


_BLK = 1000


def _mm_body(x_ref, w_ref, o_ref):
    o_ref[...] = jnp.dot(x_ref[...], w_ref[...],
                         preferred_element_type=jnp.float32)


def _matmul(x, W):
    M, K = x.shape
    Dout = W.shape[1]
    return pl.pallas_call(
        _mm_body,
        grid=(M // _BLK,),
        in_specs=[pl.BlockSpec((_BLK, K), lambda i: (i, 0)),
                  pl.BlockSpec((K, Dout), lambda i: (0, 0))],
        out_specs=pl.BlockSpec((_BLK, Dout), lambda i: (i, 0)),
        out_shape=jax.ShapeDtypeStruct((M, Dout), jnp.float32),
    )(x, W)


def _cmm_body(p_ref, b_ref, w_ref, act_ref, sup_ref):
    act = jnp.maximum(p_ref[0] + p_ref[1] + b_ref[...], 0.0)
    act_ref[...] = act
    sup_ref[...] = jnp.dot(act, w_ref[...], preferred_element_type=jnp.float32)


def _combine_mm(P, b, W):
    D = P.shape[2]
    Dout = W.shape[1]
    return pl.pallas_call(
        _cmm_body,
        grid=(N // _BLK,),
        in_specs=[pl.BlockSpec((2, _BLK, D), lambda i: (0, i, 0)),
                  pl.BlockSpec((1, D), lambda i: (0, 0)),
                  pl.BlockSpec((D, Dout), lambda i: (0, 0))],
        out_specs=[pl.BlockSpec((_BLK, D), lambda i: (i, 0)),
                   pl.BlockSpec((_BLK, Dout), lambda i: (i, 0))],
        out_shape=[jax.ShapeDtypeStruct((N, D), jnp.float32),
                   jax.ShapeDtypeStruct((N, Dout), jnp.float32)],
    )(P, b.reshape(1, D), W)


def _c2_body(pa_ref, pb_ref, b_ref, act_ref):
    act = jnp.concatenate([pa_ref[0] + pa_ref[1], pb_ref[0] + pb_ref[1]],
                          axis=1)
    act_ref[...] = jnp.maximum(act + b_ref[...], 0.0)


def _combine2(Pa, Pb, b):
    Dh = Pa.shape[2]
    D = 2 * Dh
    return pl.pallas_call(
        _c2_body,
        grid=(N // _BLK,),
        in_specs=[pl.BlockSpec((2, _BLK, Dh), lambda i: (0, i, 0)),
                  pl.BlockSpec((2, _BLK, Dh), lambda i: (0, i, 0)),
                  pl.BlockSpec((1, D), lambda i: (0, 0))],
        out_specs=pl.BlockSpec((_BLK, D), lambda i: (i, 0)),
        out_shape=jax.ShapeDtypeStruct((N, D), jnp.float32),
    )(Pa, Pb, b.reshape(1, D))


def _branch(feat, ew, ei, Ws, bs):
    pad = ((0, 0), (0, EPT_P - EPT))
    dst = jnp.pad(ei[0].astype(jnp.int32).reshape(NW, EPT),
                  pad).reshape(NW, NCHUNK, C)
    src = jnp.pad(ei[1].astype(jnp.int32).reshape(NW, EPT),
                  pad).reshape(NW, NCHUNK, C)
    ew = jnp.pad(ew.reshape(NW, EPT), pad).reshape(NW, NCHUNK, C)
    s0 = _matmul(feat, Ws[0])
    P = _make_spmm(Ws[0].shape[1])(dst, src, ew, s0)
    _, s1 = _combine_mm(P, bs[0], Ws[1])
    P = _make_spmm(Ws[1].shape[1])(dst, src, ew, s1)
    p0, s2 = _combine_mm(P, bs[1], Ws[2])
    P = _make_spmm(Ws[2].shape[1])(dst, src, ew, s2)
    _, s3 = _combine_mm(P, bs[2], Ws[3])
    Dh = Ws[3].shape[1] // 2
    Pa = _make_spmm(Dh)(dst, src, ew, s3[:, :Dh])
    Pb = _make_spmm(Dh)(dst, src, ew, s3[:, Dh:])
    h1 = _combine2(Pa, Pb, bs[3])
    return p0, h1


def kernel(feature1, edge_weight1, feature2, edge_weight2, params,
           edge_index1, edge_index2):
    p0, h1 = _branch(feature1, edge_weight1, edge_index1,
                     params["Ws1"], params["bs1"])
    p3, h4 = _branch(feature2, edge_weight2, edge_index2,
                     params["Ws2"], params["bs2"])
    return (p0, h1, p3, h4)


# speedup vs baseline: 1.6685x; 1.1970x over previous
import functools

import jax
import jax.numpy as jnp
from jax import lax
from jax.experimental import pallas as pl
from jax.experimental.pallas import tpu as pltpu
from jax.experimental.pallas import tpu_sc as plsc

N = 10000
E = 320000
NC, NS, L = 2, 16, 16
NW = NC * NS
EPT = E // NW
C = 80
NCHUNK = 125
EPT_P = NCHUNK * C
NP = 10240
RPT = NP // NS
RC = 80
NRC = RPT // RC


def _vreg_gather(v, idx):
    return lax.gather(
        v, idx[:, None],
        dimension_numbers=lax.GatherDimensionNumbers(
            offset_dims=(), collapsed_slice_dims=(0,), start_index_map=(0,)),
        slice_sizes=(1,),
        mode=lax.GatherScatterMode.PROMISE_IN_BOUNDS)


def _spmm_body(D, dst_h, src_h, w_h, sup_h, out_h,
               acc_sh, dst_v, src_v, w_v,
               gbuf0, gbuf1, gbuf2, sbuf0, sbuf1, sbuf2,
               gsem0, gsem1, gsem2, ssem0, ssem1, ssem2):
    c = lax.axis_index("c")
    s = lax.axis_index("s")
    wid = c * NS + s
    rbase = s * RPT
    zero16 = jnp.zeros((L,), jnp.float32)
    zi16 = jnp.zeros((L,), jnp.int32)

    @pl.loop(0, RC)
    def _zero_row(i):
        for j in range(D // L):
            gbuf0[i, pl.ds(j * L, L)] = zero16

    for r in range(NRC):
        pltpu.async_copy(gbuf0.at[pl.ds(0, RC)],
                         acc_sh.at[pl.ds(rbase + r * RC, RC)], gsem0)

    pltpu.async_copy(dst_h.at[wid], dst_v, gsem1)
    pltpu.async_copy(src_h.at[wid], src_v, gsem1)
    pltpu.async_copy(w_h.at[wid], w_v, gsem1)
    pltpu.make_async_copy(dst_h.at[wid], dst_v, gsem1).wait()
    pltpu.make_async_copy(src_h.at[wid], src_v, gsem1).wait()
    pltpu.make_async_copy(w_h.at[wid], w_v, gsem1).wait()
    for r in range(NRC):
        pltpu.make_async_copy(gbuf0.at[pl.ds(0, RC)],
                              acc_sh.at[pl.ds(rbase + r * RC, RC)],
                              gsem0).wait()

    pltpu.async_copy(sup_h.at[src_v.at[0]], gbuf0, gsem0)
    pltpu.async_copy(sup_h.at[src_v.at[1]], gbuf1, gsem1)
    pltpu.async_copy(sup_h.at[src_v.at[2]], gbuf2, gsem2)
    plsc.subcore_barrier()

    def process(j, gbuf, sbuf, gsem, ssem, refill, waitprev):
        pltpu.make_async_copy(sup_h.at[src_v.at[j]], gbuf, gsem).wait()

        @pl.loop(0, C // L)
        def _scale(g):
            w16 = w_v[j, pl.ds(g * L, L)]
            for ii in range(L):
                wb = _vreg_gather(w16, jnp.full((L,), ii, jnp.int32))
                i = g * L + ii
                for d in range(D // L):
                    sl = pl.ds(d * L, L)
                    sbuf[i, sl] = gbuf[i, sl] * wb

        if refill:
            pltpu.async_copy(sup_h.at[src_v.at[j + 3]], gbuf, gsem)

        if waitprev:
            pltpu.make_async_copy(sbuf, acc_sh.at[dst_v.at[j - 3]],
                                  ssem).wait()
        pltpu.async_copy(sbuf, acc_sh.at[dst_v.at[j]], ssem, add=True)

    B0 = (gbuf0, sbuf0, gsem0, ssem0)
    B1 = (gbuf1, sbuf1, gsem1, ssem1)
    B2 = (gbuf2, sbuf2, gsem2, ssem2)
    process(0, *B0, True, False)
    process(1, *B1, True, False)
    process(2, *B2, True, False)

    @pl.loop(3, NCHUNK - 5, step=3)
    def _trip(k):
        process(k, *B0, True, True)
        process(k + 1, *B1, True, True)
        process(k + 2, *B2, True, True)

    process(NCHUNK - 5, *B0, True, True)
    process(NCHUNK - 4, *B1, True, True)
    process(NCHUNK - 3, *B2, False, True)
    process(NCHUNK - 2, *B0, False, True)
    process(NCHUNK - 1, *B1, False, True)
    pltpu.make_async_copy(sbuf2, acc_sh.at[dst_v.at[NCHUNK - 3]],
                          ssem2).wait()
    pltpu.make_async_copy(sbuf0, acc_sh.at[dst_v.at[NCHUNK - 2]],
                          ssem0).wait()
    pltpu.make_async_copy(sbuf1, acc_sh.at[dst_v.at[NCHUNK - 1]],
                          ssem1).wait()
    plsc.subcore_barrier()

    for r in range(NRC):
        pltpu.async_copy(acc_sh.at[pl.ds(rbase + r * RC, RC)],
                         out_h.at[c, pl.ds(rbase + r * RC, RC)], gsem0)
    for r in range(NRC):
        pltpu.make_async_copy(acc_sh.at[pl.ds(rbase + r * RC, RC)],
                              out_h.at[c, pl.ds(rbase + r * RC, RC)],
                              gsem0).wait()


@functools.cache
def _make_spmm(D):
    mesh = plsc.VectorSubcoreMesh(core_axis_name="c", subcore_axis_name="s",
                                  num_cores=NC, num_subcores=NS)
    return pl.kernel(
        functools.partial(_spmm_body, D),
        out_type=jax.ShapeDtypeStruct((NC, NP, D), jnp.float32),
        mesh=mesh,
        scratch_types=[
            pltpu.VMEM_SHARED((NP, D), jnp.float32),
            pltpu.VMEM((NCHUNK, C), jnp.int32),
            pltpu.VMEM((NCHUNK, C), jnp.int32),
            pltpu.VMEM((NCHUNK, C), jnp.float32),
            pltpu.VMEM((C, D), jnp.float32),
            pltpu.VMEM((C, D), jnp.float32),
            pltpu.VMEM((C, D), jnp.float32),
            pltpu.VMEM((C, D), jnp.float32),
            pltpu.VMEM((C, D), jnp.float32),
            pltpu.VMEM((C, D), jnp.float32),
            pltpu.SemaphoreType.DMA,
            pltpu.SemaphoreType.DMA,
            pltpu.SemaphoreType.DMA,
            pltpu.SemaphoreType.DMA,
            pltpu.SemaphoreType.DMA,
            pltpu.SemaphoreType.DMA,
        ],
        compiler_params=pltpu.CompilerParams(needs_layout_passes=False,
                                             use_tc_tiling_on_sc=False),
    )


_BLK = 1000


def _mm_body(x_ref, w_ref, o_ref):
    o_ref[...] = jnp.dot(x_ref[...], w_ref[...],
                         preferred_element_type=jnp.float32)


def _matmul(x, W):
    M, K = x.shape
    Dout = W.shape[1]
    return pl.pallas_call(
        _mm_body,
        grid=(M // _BLK,),
        in_specs=[pl.BlockSpec((_BLK, K), lambda i: (i, 0)),
                  pl.BlockSpec((K, Dout), lambda i: (0, 0))],
        out_specs=pl.BlockSpec((_BLK, Dout), lambda i: (i, 0)),
        out_shape=jax.ShapeDtypeStruct((M, Dout), jnp.float32),
    )(x, W)


def _cmm_body(p_ref, b_ref, w_ref, act_ref, sup_ref):
    act = jnp.maximum(p_ref[0] + p_ref[1] + b_ref[...], 0.0)
    act_ref[...] = act
    sup_ref[...] = jnp.dot(act, w_ref[...], preferred_element_type=jnp.float32)


def _combine_mm(P, b, W):
    D = P.shape[2]
    Dout = W.shape[1]
    return pl.pallas_call(
        _cmm_body,
        grid=(N // _BLK,),
        in_specs=[pl.BlockSpec((2, _BLK, D), lambda i: (0, i, 0)),
                  pl.BlockSpec((1, D), lambda i: (0, 0)),
                  pl.BlockSpec((D, Dout), lambda i: (0, 0))],
        out_specs=[pl.BlockSpec((_BLK, D), lambda i: (i, 0)),
                   pl.BlockSpec((_BLK, Dout), lambda i: (i, 0))],
        out_shape=[jax.ShapeDtypeStruct((N, D), jnp.float32),
                   jax.ShapeDtypeStruct((N, Dout), jnp.float32)],
    )(P, b.reshape(1, D), W)


def _c2_body(pa_ref, pb_ref, b_ref, act_ref):
    act = jnp.concatenate([pa_ref[0] + pa_ref[1], pb_ref[0] + pb_ref[1]],
                          axis=1)
    act_ref[...] = jnp.maximum(act + b_ref[...], 0.0)


def _combine2(Pa, Pb, b):
    Dh = Pa.shape[2]
    D = 2 * Dh
    return pl.pallas_call(
        _c2_body,
        grid=(N // _BLK,),
        in_specs=[pl.BlockSpec((2, _BLK, Dh), lambda i: (0, i, 0)),
                  pl.BlockSpec((2, _BLK, Dh), lambda i: (0, i, 0)),
                  pl.BlockSpec((1, D), lambda i: (0, 0))],
        out_specs=pl.BlockSpec((_BLK, D), lambda i: (i, 0)),
        out_shape=jax.ShapeDtypeStruct((N, D), jnp.float32),
    )(Pa, Pb, b.reshape(1, D))


def _branch(feat, ew, ei, Ws, bs):
    pad = ((0, 0), (0, EPT_P - EPT))
    dst = jnp.pad(ei[0].astype(jnp.int32).reshape(NW, EPT),
                  pad).reshape(NW, NCHUNK, C)
    src = jnp.pad(ei[1].astype(jnp.int32).reshape(NW, EPT),
                  pad).reshape(NW, NCHUNK, C)
    ew = jnp.pad(ew.reshape(NW, EPT), pad).reshape(NW, NCHUNK, C)
    s0 = _matmul(feat, Ws[0])
    P = _make_spmm(Ws[0].shape[1])(dst, src, ew, s0)
    _, s1 = _combine_mm(P, bs[0], Ws[1])
    P = _make_spmm(Ws[1].shape[1])(dst, src, ew, s1)
    p0, s2 = _combine_mm(P, bs[1], Ws[2])
    P = _make_spmm(Ws[2].shape[1])(dst, src, ew, s2)
    _, s3 = _combine_mm(P, bs[2], Ws[3])
    Dh = Ws[3].shape[1] // 2
    Pa = _make_spmm(Dh)(dst, src, ew, s3[:, :Dh])
    Pb = _make_spmm(Dh)(dst, src, ew, s3[:, Dh:])
    h1 = _combine2(Pa, Pb, bs[3])
    return p0, h1


def kernel(feature1, edge_weight1, feature2, edge_weight2, params,
           edge_index1, edge_index2):
    p0, h1 = _branch(feature1, edge_weight1, edge_index1,
                     params["Ws1"], params["bs1"])
    p3, h4 = _branch(feature2, edge_weight2, edge_index2,
                     params["Ws2"], params["bs2"])
    return (p0, h1, p3, h4)


# speedup vs baseline: 1.7190x; 1.0303x over previous
import functools

import jax
import jax.numpy as jnp
from jax import lax
from jax.experimental import pallas as pl
from jax.experimental.pallas import tpu as pltpu
from jax.experimental.pallas import tpu_sc as plsc

N = 10000
E = 320000
NC, NS, L = 2, 16, 16
NW = NC * NS
EPT = E // NW
C = 80
NCHUNK = 125
EPT_P = NCHUNK * C
NP = 10240
RPT = NP // NS
RC = 80
NRC = RPT // RC


def _vreg_gather(v, idx):
    return lax.gather(
        v, idx[:, None],
        dimension_numbers=lax.GatherDimensionNumbers(
            offset_dims=(), collapsed_slice_dims=(0,), start_index_map=(0,)),
        slice_sizes=(1,),
        mode=lax.GatherScatterMode.PROMISE_IN_BOUNDS)


def _spmm_body(D, dst_h, src_h, w_h, sup_h, out_h,
               acc_sh, dst_v, src_v, w_v,
               gbuf0, gbuf1, gbuf2, gbuf3, sbuf0, sbuf1, sbuf2, sbuf3,
               gsem0, gsem1, gsem2, gsem3, ssem0, ssem1, ssem2, ssem3):
    c = lax.axis_index("c")
    s = lax.axis_index("s")
    wid = c * NS + s
    rbase = s * RPT
    zero16 = jnp.zeros((L,), jnp.float32)
    zi16 = jnp.zeros((L,), jnp.int32)

    @pl.loop(0, RC)
    def _zero_row(i):
        for j in range(D // L):
            gbuf0[i, pl.ds(j * L, L)] = zero16

    for r in range(NRC):
        pltpu.async_copy(gbuf0.at[pl.ds(0, RC)],
                         acc_sh.at[pl.ds(rbase + r * RC, RC)], gsem0)

    pltpu.async_copy(dst_h.at[wid], dst_v, gsem1)
    pltpu.async_copy(src_h.at[wid], src_v, gsem1)
    pltpu.async_copy(w_h.at[wid], w_v, gsem1)
    pltpu.make_async_copy(dst_h.at[wid], dst_v, gsem1).wait()
    pltpu.make_async_copy(src_h.at[wid], src_v, gsem1).wait()
    pltpu.make_async_copy(w_h.at[wid], w_v, gsem1).wait()
    for r in range(NRC):
        pltpu.make_async_copy(gbuf0.at[pl.ds(0, RC)],
                              acc_sh.at[pl.ds(rbase + r * RC, RC)],
                              gsem0).wait()

    pltpu.async_copy(sup_h.at[src_v.at[0]], gbuf0, gsem0)
    pltpu.async_copy(sup_h.at[src_v.at[1]], gbuf1, gsem1)
    pltpu.async_copy(sup_h.at[src_v.at[2]], gbuf2, gsem2)
    pltpu.async_copy(sup_h.at[src_v.at[3]], gbuf3, gsem3)
    plsc.subcore_barrier()

    def process(j, gbuf, sbuf, gsem, ssem, refill, waitprev):
        pltpu.make_async_copy(sup_h.at[src_v.at[j]], gbuf, gsem).wait()

        @pl.loop(0, C // L)
        def _scale(g):
            w16 = w_v[j, pl.ds(g * L, L)]
            for ii in range(L):
                wb = _vreg_gather(w16, jnp.full((L,), ii, jnp.int32))
                i = g * L + ii
                for d in range(D // L):
                    sl = pl.ds(d * L, L)
                    sbuf[i, sl] = gbuf[i, sl] * wb

        if refill:
            pltpu.async_copy(sup_h.at[src_v.at[j + 4]], gbuf, gsem)

        if waitprev:
            pltpu.make_async_copy(sbuf, acc_sh.at[dst_v.at[j - 4]],
                                  ssem).wait()
        pltpu.async_copy(sbuf, acc_sh.at[dst_v.at[j]], ssem, add=True)

    B0 = (gbuf0, sbuf0, gsem0, ssem0)
    B1 = (gbuf1, sbuf1, gsem1, ssem1)
    B2 = (gbuf2, sbuf2, gsem2, ssem2)
    B3 = (gbuf3, sbuf3, gsem3, ssem3)
    process(0, *B0, True, False)
    process(1, *B1, True, False)
    process(2, *B2, True, False)
    process(3, *B3, True, False)

    @pl.loop(4, NCHUNK - 5, step=4)
    def _quad(k):
        process(k, *B0, True, True)
        process(k + 1, *B1, True, True)
        process(k + 2, *B2, True, True)
        process(k + 3, *B3, True, True)

    process(NCHUNK - 5, *B0, True, True)
    process(NCHUNK - 4, *B1, False, True)
    process(NCHUNK - 3, *B2, False, True)
    process(NCHUNK - 2, *B3, False, True)
    process(NCHUNK - 1, *B0, False, True)
    pltpu.make_async_copy(sbuf1, acc_sh.at[dst_v.at[NCHUNK - 4]],
                          ssem1).wait()
    pltpu.make_async_copy(sbuf2, acc_sh.at[dst_v.at[NCHUNK - 3]],
                          ssem2).wait()
    pltpu.make_async_copy(sbuf3, acc_sh.at[dst_v.at[NCHUNK - 2]],
                          ssem3).wait()
    pltpu.make_async_copy(sbuf0, acc_sh.at[dst_v.at[NCHUNK - 1]],
                          ssem0).wait()
    plsc.subcore_barrier()

    for r in range(NRC):
        pltpu.async_copy(acc_sh.at[pl.ds(rbase + r * RC, RC)],
                         out_h.at[c, pl.ds(rbase + r * RC, RC)], gsem0)
    for r in range(NRC):
        pltpu.make_async_copy(acc_sh.at[pl.ds(rbase + r * RC, RC)],
                              out_h.at[c, pl.ds(rbase + r * RC, RC)],
                              gsem0).wait()


@functools.cache
def _make_spmm(D):
    mesh = plsc.VectorSubcoreMesh(core_axis_name="c", subcore_axis_name="s",
                                  num_cores=NC, num_subcores=NS)
    return pl.kernel(
        functools.partial(_spmm_body, D),
        out_type=jax.ShapeDtypeStruct((NC, NP, D), jnp.float32),
        mesh=mesh,
        scratch_types=[
            pltpu.VMEM_SHARED((NP, D), jnp.float32),
            pltpu.VMEM((NCHUNK, C), jnp.int32),
            pltpu.VMEM((NCHUNK, C), jnp.int32),
            pltpu.VMEM((NCHUNK, C), jnp.float32),
        ] + [pltpu.VMEM((C, D), jnp.float32)] * 8
          + [pltpu.SemaphoreType.DMA] * 8,
        compiler_params=pltpu.CompilerParams(needs_layout_passes=False,
                                             use_tc_tiling_on_sc=False),
    )


_BLK = 1000


def _mm_body(x_ref, w_ref, o_ref):
    o_ref[...] = jnp.dot(x_ref[...], w_ref[...],
                         preferred_element_type=jnp.float32)


def _matmul(x, W):
    M, K = x.shape
    Dout = W.shape[1]
    return pl.pallas_call(
        _mm_body,
        grid=(M // _BLK,),
        in_specs=[pl.BlockSpec((_BLK, K), lambda i: (i, 0)),
                  pl.BlockSpec((K, Dout), lambda i: (0, 0))],
        out_specs=pl.BlockSpec((_BLK, Dout), lambda i: (i, 0)),
        out_shape=jax.ShapeDtypeStruct((M, Dout), jnp.float32),
    )(x, W)


def _cmm_body(p_ref, b_ref, w_ref, act_ref, sup_ref):
    act = jnp.maximum(p_ref[0] + p_ref[1] + b_ref[...], 0.0)
    act_ref[...] = act
    sup_ref[...] = jnp.dot(act, w_ref[...], preferred_element_type=jnp.float32)


def _combine_mm(P, b, W):
    D = P.shape[2]
    Dout = W.shape[1]
    return pl.pallas_call(
        _cmm_body,
        grid=(N // _BLK,),
        in_specs=[pl.BlockSpec((2, _BLK, D), lambda i: (0, i, 0)),
                  pl.BlockSpec((1, D), lambda i: (0, 0)),
                  pl.BlockSpec((D, Dout), lambda i: (0, 0))],
        out_specs=[pl.BlockSpec((_BLK, D), lambda i: (i, 0)),
                   pl.BlockSpec((_BLK, Dout), lambda i: (i, 0))],
        out_shape=[jax.ShapeDtypeStruct((N, D), jnp.float32),
                   jax.ShapeDtypeStruct((N, Dout), jnp.float32)],
    )(P, b.reshape(1, D), W)


def _c2_body(pa_ref, pb_ref, b_ref, act_ref):
    act = jnp.concatenate([pa_ref[0] + pa_ref[1], pb_ref[0] + pb_ref[1]],
                          axis=1)
    act_ref[...] = jnp.maximum(act + b_ref[...], 0.0)


def _combine2(Pa, Pb, b):
    Dh = Pa.shape[2]
    D = 2 * Dh
    return pl.pallas_call(
        _c2_body,
        grid=(N // _BLK,),
        in_specs=[pl.BlockSpec((2, _BLK, Dh), lambda i: (0, i, 0)),
                  pl.BlockSpec((2, _BLK, Dh), lambda i: (0, i, 0)),
                  pl.BlockSpec((1, D), lambda i: (0, 0))],
        out_specs=pl.BlockSpec((_BLK, D), lambda i: (i, 0)),
        out_shape=jax.ShapeDtypeStruct((N, D), jnp.float32),
    )(Pa, Pb, b.reshape(1, D))


def _branch(feat, ew, ei, Ws, bs):
    pad = ((0, 0), (0, EPT_P - EPT))
    dst = jnp.pad(ei[0].astype(jnp.int32).reshape(NW, EPT),
                  pad).reshape(NW, NCHUNK, C)
    src = jnp.pad(ei[1].astype(jnp.int32).reshape(NW, EPT),
                  pad).reshape(NW, NCHUNK, C)
    ew = jnp.pad(ew.reshape(NW, EPT), pad).reshape(NW, NCHUNK, C)
    s0 = _matmul(feat, Ws[0])
    P = _make_spmm(Ws[0].shape[1])(dst, src, ew, s0)
    _, s1 = _combine_mm(P, bs[0], Ws[1])
    P = _make_spmm(Ws[1].shape[1])(dst, src, ew, s1)
    p0, s2 = _combine_mm(P, bs[1], Ws[2])
    P = _make_spmm(Ws[2].shape[1])(dst, src, ew, s2)
    _, s3 = _combine_mm(P, bs[2], Ws[3])
    Dh = Ws[3].shape[1] // 2
    Pa = _make_spmm(Dh)(dst, src, ew, s3[:, :Dh])
    Pb = _make_spmm(Dh)(dst, src, ew, s3[:, Dh:])
    h1 = _combine2(Pa, Pb, bs[3])
    return p0, h1


def kernel(feature1, edge_weight1, feature2, edge_weight2, params,
           edge_index1, edge_index2):
    p0, h1 = _branch(feature1, edge_weight1, edge_index1,
                     params["Ws1"], params["bs1"])
    p3, h4 = _branch(feature2, edge_weight2, edge_index2,
                     params["Ws2"], params["bs2"])
    return (p0, h1, p3, h4)


# speedup vs baseline: 1.7218x; 1.0016x over previous
import functools

import jax
import jax.numpy as jnp
from jax import lax
from jax.experimental import pallas as pl
from jax.experimental.pallas import tpu as pltpu
from jax.experimental.pallas import tpu_sc as plsc

N = 10000
E = 320000
NC, NS, L = 2, 16, 16
NW = NC * NS
EPT = E // NW
C = 80
NCHUNK = 125
EPT_P = NCHUNK * C
NP = 10240
RPT = NP // NS
RC = 80
NRC = RPT // RC


def _vreg_gather(v, idx):
    return lax.gather(
        v, idx[:, None],
        dimension_numbers=lax.GatherDimensionNumbers(
            offset_dims=(), collapsed_slice_dims=(0,), start_index_map=(0,)),
        slice_sizes=(1,),
        mode=lax.GatherScatterMode.PROMISE_IN_BOUNDS)


def _spmm_body(D, dst_h, src_h, w_h, sup_h, out_h,
               acc_sh, dst_v, src_v, w_v,
               gbuf0, gbuf1, gbuf2, gbuf3, sbuf0, sbuf1, sbuf2, sbuf3,
               gsem0, gsem1, gsem2, gsem3, ssem0, ssem1, ssem2, ssem3):
    c = lax.axis_index("c")
    s = lax.axis_index("s")
    wid = c * NS + s
    rbase = s * RPT
    zero16 = jnp.zeros((L,), jnp.float32)
    zi16 = jnp.zeros((L,), jnp.int32)

    @pl.loop(0, RC)
    def _zero_row(i):
        for j in range(D // L):
            gbuf0[i, pl.ds(j * L, L)] = zero16

    for r in range(NRC):
        pltpu.async_copy(gbuf0.at[pl.ds(0, RC)],
                         acc_sh.at[pl.ds(rbase + r * RC, RC)], gsem0)

    pltpu.async_copy(dst_h.at[wid], dst_v, gsem1)
    pltpu.async_copy(src_h.at[wid], src_v, gsem1)
    pltpu.async_copy(w_h.at[wid], w_v, gsem1)
    pltpu.make_async_copy(dst_h.at[wid], dst_v, gsem1).wait()
    pltpu.make_async_copy(src_h.at[wid], src_v, gsem1).wait()
    pltpu.make_async_copy(w_h.at[wid], w_v, gsem1).wait()
    for r in range(NRC):
        pltpu.make_async_copy(gbuf0.at[pl.ds(0, RC)],
                              acc_sh.at[pl.ds(rbase + r * RC, RC)],
                              gsem0).wait()

    pltpu.async_copy(sup_h.at[src_v.at[0]], gbuf0, gsem0)
    pltpu.async_copy(sup_h.at[src_v.at[1]], gbuf1, gsem1)
    pltpu.async_copy(sup_h.at[src_v.at[2]], gbuf2, gsem2)
    pltpu.async_copy(sup_h.at[src_v.at[3]], gbuf3, gsem3)
    plsc.subcore_barrier()

    def process(j, gbuf, sbuf, gsem, ssem, refill, waitprev):
        if waitprev:
            pltpu.make_async_copy(sbuf, acc_sh.at[dst_v.at[j - 4]],
                                  ssem).wait()
        pltpu.make_async_copy(sup_h.at[src_v.at[j]], gbuf, gsem).wait()

        @pl.loop(0, C // L)
        def _scale(g):
            w16 = w_v[j, pl.ds(g * L, L)]
            for ii in range(L):
                wb = _vreg_gather(w16, jnp.full((L,), ii, jnp.int32))
                i = g * L + ii
                for d in range(D // L):
                    sl = pl.ds(d * L, L)
                    sbuf[i, sl] = gbuf[i, sl] * wb

        if refill:
            pltpu.async_copy(sup_h.at[src_v.at[j + 4]], gbuf, gsem)

        pltpu.async_copy(sbuf, acc_sh.at[dst_v.at[j]], ssem, add=True)

    B0 = (gbuf0, sbuf0, gsem0, ssem0)
    B1 = (gbuf1, sbuf1, gsem1, ssem1)
    B2 = (gbuf2, sbuf2, gsem2, ssem2)
    B3 = (gbuf3, sbuf3, gsem3, ssem3)
    process(0, *B0, True, False)
    process(1, *B1, True, False)
    process(2, *B2, True, False)
    process(3, *B3, True, False)

    @pl.loop(4, NCHUNK - 5, step=4)
    def _quad(k):
        process(k, *B0, True, True)
        process(k + 1, *B1, True, True)
        process(k + 2, *B2, True, True)
        process(k + 3, *B3, True, True)

    process(NCHUNK - 5, *B0, True, True)
    process(NCHUNK - 4, *B1, False, True)
    process(NCHUNK - 3, *B2, False, True)
    process(NCHUNK - 2, *B3, False, True)
    process(NCHUNK - 1, *B0, False, True)
    pltpu.make_async_copy(sbuf1, acc_sh.at[dst_v.at[NCHUNK - 4]],
                          ssem1).wait()
    pltpu.make_async_copy(sbuf2, acc_sh.at[dst_v.at[NCHUNK - 3]],
                          ssem2).wait()
    pltpu.make_async_copy(sbuf3, acc_sh.at[dst_v.at[NCHUNK - 2]],
                          ssem3).wait()
    pltpu.make_async_copy(sbuf0, acc_sh.at[dst_v.at[NCHUNK - 1]],
                          ssem0).wait()
    plsc.subcore_barrier()

    for r in range(NRC):
        pltpu.async_copy(acc_sh.at[pl.ds(rbase + r * RC, RC)],
                         out_h.at[c, pl.ds(rbase + r * RC, RC)], gsem0)
    for r in range(NRC):
        pltpu.make_async_copy(acc_sh.at[pl.ds(rbase + r * RC, RC)],
                              out_h.at[c, pl.ds(rbase + r * RC, RC)],
                              gsem0).wait()


@functools.cache
def _make_spmm(D):
    mesh = plsc.VectorSubcoreMesh(core_axis_name="c", subcore_axis_name="s",
                                  num_cores=NC, num_subcores=NS)
    return pl.kernel(
        functools.partial(_spmm_body, D),
        out_type=jax.ShapeDtypeStruct((NC, NP, D), jnp.float32),
        mesh=mesh,
        scratch_types=[
            pltpu.VMEM_SHARED((NP, D), jnp.float32),
            pltpu.VMEM((NCHUNK, C), jnp.int32),
            pltpu.VMEM((NCHUNK, C), jnp.int32),
            pltpu.VMEM((NCHUNK, C), jnp.float32),
        ] + [pltpu.VMEM((C, D), jnp.float32)] * 8
          + [pltpu.SemaphoreType.DMA] * 8,
        compiler_params=pltpu.CompilerParams(needs_layout_passes=False,
                                             use_tc_tiling_on_sc=False),
    )


_BLK = 1000


def _mm_body(x_ref, w_ref, o_ref):
    o_ref[...] = jnp.dot(x_ref[...], w_ref[...],
                         preferred_element_type=jnp.float32)


def _matmul(x, W):
    M, K = x.shape
    Dout = W.shape[1]
    return pl.pallas_call(
        _mm_body,
        grid=(M // _BLK,),
        in_specs=[pl.BlockSpec((_BLK, K), lambda i: (i, 0)),
                  pl.BlockSpec((K, Dout), lambda i: (0, 0))],
        out_specs=pl.BlockSpec((_BLK, Dout), lambda i: (i, 0)),
        out_shape=jax.ShapeDtypeStruct((M, Dout), jnp.float32),
    )(x, W)


def _cmm_body(p_ref, b_ref, w_ref, act_ref, sup_ref):
    act = jnp.maximum(p_ref[0] + p_ref[1] + b_ref[...], 0.0)
    act_ref[...] = act
    sup_ref[...] = jnp.dot(act, w_ref[...], preferred_element_type=jnp.float32)


def _combine_mm(P, b, W):
    D = P.shape[2]
    Dout = W.shape[1]
    return pl.pallas_call(
        _cmm_body,
        grid=(N // _BLK,),
        in_specs=[pl.BlockSpec((2, _BLK, D), lambda i: (0, i, 0)),
                  pl.BlockSpec((1, D), lambda i: (0, 0)),
                  pl.BlockSpec((D, Dout), lambda i: (0, 0))],
        out_specs=[pl.BlockSpec((_BLK, D), lambda i: (i, 0)),
                   pl.BlockSpec((_BLK, Dout), lambda i: (i, 0))],
        out_shape=[jax.ShapeDtypeStruct((N, D), jnp.float32),
                   jax.ShapeDtypeStruct((N, Dout), jnp.float32)],
    )(P, b.reshape(1, D), W)


def _c2_body(pa_ref, pb_ref, b_ref, act_ref):
    act = jnp.concatenate([pa_ref[0] + pa_ref[1], pb_ref[0] + pb_ref[1]],
                          axis=1)
    act_ref[...] = jnp.maximum(act + b_ref[...], 0.0)


def _combine2(Pa, Pb, b):
    Dh = Pa.shape[2]
    D = 2 * Dh
    return pl.pallas_call(
        _c2_body,
        grid=(N // _BLK,),
        in_specs=[pl.BlockSpec((2, _BLK, Dh), lambda i: (0, i, 0)),
                  pl.BlockSpec((2, _BLK, Dh), lambda i: (0, i, 0)),
                  pl.BlockSpec((1, D), lambda i: (0, 0))],
        out_specs=pl.BlockSpec((_BLK, D), lambda i: (i, 0)),
        out_shape=jax.ShapeDtypeStruct((N, D), jnp.float32),
    )(Pa, Pb, b.reshape(1, D))


def _branch(feat, ew, ei, Ws, bs):
    pad = ((0, 0), (0, EPT_P - EPT))
    dst = jnp.pad(ei[0].astype(jnp.int32).reshape(NW, EPT),
                  pad).reshape(NW, NCHUNK, C)
    src = jnp.pad(ei[1].astype(jnp.int32).reshape(NW, EPT),
                  pad).reshape(NW, NCHUNK, C)
    ew = jnp.pad(ew.reshape(NW, EPT), pad).reshape(NW, NCHUNK, C)
    s0 = _matmul(feat, Ws[0])
    P = _make_spmm(Ws[0].shape[1])(dst, src, ew, s0)
    _, s1 = _combine_mm(P, bs[0], Ws[1])
    P = _make_spmm(Ws[1].shape[1])(dst, src, ew, s1)
    p0, s2 = _combine_mm(P, bs[1], Ws[2])
    P = _make_spmm(Ws[2].shape[1])(dst, src, ew, s2)
    _, s3 = _combine_mm(P, bs[2], Ws[3])
    Dh = Ws[3].shape[1] // 2
    Pa = _make_spmm(Dh)(dst, src, ew, s3[:, :Dh])
    Pb = _make_spmm(Dh)(dst, src, ew, s3[:, Dh:])
    h1 = _combine2(Pa, Pb, bs[3])
    return p0, h1


def kernel(feature1, edge_weight1, feature2, edge_weight2, params,
           edge_index1, edge_index2):
    p0, h1 = _branch(feature1, edge_weight1, edge_index1,
                     params["Ws1"], params["bs1"])
    p3, h4 = _branch(feature2, edge_weight2, edge_index2,
                     params["Ws2"], params["bs2"])
    return (p0, h1, p3, h4)


# speedup vs baseline: 1.7250x; 1.0018x over previous
import functools

import jax
import jax.numpy as jnp
from jax import lax
from jax.experimental import pallas as pl
from jax.experimental.pallas import tpu as pltpu
from jax.experimental.pallas import tpu_sc as plsc

N = 10000
E = 320000
NC, NS, L = 2, 16, 16
NW = NC * NS
EPT = E // NW
C = 80
NCHUNK = 125
EPT_P = NCHUNK * C
NP = 10240
RPT = NP // NS
RC = 80
NRC = RPT // RC


def _vreg_gather(v, idx):
    return lax.gather(
        v, idx[:, None],
        dimension_numbers=lax.GatherDimensionNumbers(
            offset_dims=(), collapsed_slice_dims=(0,), start_index_map=(0,)),
        slice_sizes=(1,),
        mode=lax.GatherScatterMode.PROMISE_IN_BOUNDS)


def _spmm_body(D, dst_h, src_h, w_h, sup_h, out_h,
               acc_sh, dst_v, src_v, w_v,
               gbuf0, gbuf1, gbuf2, gbuf3, gbuf4,
               sbuf0, sbuf1, sbuf2, sbuf3, sbuf4,
               gsem0, gsem1, gsem2, gsem3, gsem4,
               ssem0, ssem1, ssem2, ssem3, ssem4):
    c = lax.axis_index("c")
    s = lax.axis_index("s")
    wid = c * NS + s
    rbase = s * RPT
    zero16 = jnp.zeros((L,), jnp.float32)

    @pl.loop(0, RC)
    def _zero_row(i):
        for j in range(D // L):
            gbuf0[i, pl.ds(j * L, L)] = zero16

    for r in range(NRC):
        pltpu.async_copy(gbuf0.at[pl.ds(0, RC)],
                         acc_sh.at[pl.ds(rbase + r * RC, RC)], gsem0)

    pltpu.async_copy(dst_h.at[wid], dst_v, gsem1)
    pltpu.async_copy(src_h.at[wid], src_v, gsem1)
    pltpu.async_copy(w_h.at[wid], w_v, gsem1)
    pltpu.make_async_copy(dst_h.at[wid], dst_v, gsem1).wait()
    pltpu.make_async_copy(src_h.at[wid], src_v, gsem1).wait()
    pltpu.make_async_copy(w_h.at[wid], w_v, gsem1).wait()
    for r in range(NRC):
        pltpu.make_async_copy(gbuf0.at[pl.ds(0, RC)],
                              acc_sh.at[pl.ds(rbase + r * RC, RC)],
                              gsem0).wait()

    pltpu.async_copy(sup_h.at[src_v.at[0]], gbuf0, gsem0)
    pltpu.async_copy(sup_h.at[src_v.at[1]], gbuf1, gsem1)
    pltpu.async_copy(sup_h.at[src_v.at[2]], gbuf2, gsem2)
    pltpu.async_copy(sup_h.at[src_v.at[3]], gbuf3, gsem3)
    pltpu.async_copy(sup_h.at[src_v.at[4]], gbuf4, gsem4)
    plsc.subcore_barrier()

    def process(j, gbuf, sbuf, gsem, ssem, refill, waitprev):
        if waitprev:
            pltpu.make_async_copy(sbuf, acc_sh.at[dst_v.at[j - 5]],
                                  ssem).wait()
        pltpu.make_async_copy(sup_h.at[src_v.at[j]], gbuf, gsem).wait()

        @pl.loop(0, C // L)
        def _scale(g):
            w16 = w_v[j, pl.ds(g * L, L)]
            for ii in range(L):
                wb = _vreg_gather(w16, jnp.full((L,), ii, jnp.int32))
                i = g * L + ii
                for d in range(D // L):
                    sl = pl.ds(d * L, L)
                    sbuf[i, sl] = gbuf[i, sl] * wb

        if refill:
            pltpu.async_copy(sup_h.at[src_v.at[j + 5]], gbuf, gsem)

        pltpu.async_copy(sbuf, acc_sh.at[dst_v.at[j]], ssem, add=True)

    B0 = (gbuf0, sbuf0, gsem0, ssem0)
    B1 = (gbuf1, sbuf1, gsem1, ssem1)
    B2 = (gbuf2, sbuf2, gsem2, ssem2)
    B3 = (gbuf3, sbuf3, gsem3, ssem3)
    B4 = (gbuf4, sbuf4, gsem4, ssem4)
    process(0, *B0, True, False)
    process(1, *B1, True, False)
    process(2, *B2, True, False)
    process(3, *B3, True, False)
    process(4, *B4, True, False)

    @pl.loop(5, NCHUNK - 5, step=5)
    def _quint(k):
        process(k, *B0, True, True)
        process(k + 1, *B1, True, True)
        process(k + 2, *B2, True, True)
        process(k + 3, *B3, True, True)
        process(k + 4, *B4, True, True)

    process(NCHUNK - 5, *B0, False, True)
    process(NCHUNK - 4, *B1, False, True)
    process(NCHUNK - 3, *B2, False, True)
    process(NCHUNK - 2, *B3, False, True)
    process(NCHUNK - 1, *B4, False, True)
    pltpu.make_async_copy(sbuf0, acc_sh.at[dst_v.at[NCHUNK - 5]],
                          ssem0).wait()
    pltpu.make_async_copy(sbuf1, acc_sh.at[dst_v.at[NCHUNK - 4]],
                          ssem1).wait()
    pltpu.make_async_copy(sbuf2, acc_sh.at[dst_v.at[NCHUNK - 3]],
                          ssem2).wait()
    pltpu.make_async_copy(sbuf3, acc_sh.at[dst_v.at[NCHUNK - 2]],
                          ssem3).wait()
    pltpu.make_async_copy(sbuf4, acc_sh.at[dst_v.at[NCHUNK - 1]],
                          ssem4).wait()
    plsc.subcore_barrier()

    for r in range(NRC):
        pltpu.async_copy(acc_sh.at[pl.ds(rbase + r * RC, RC)],
                         out_h.at[c, pl.ds(rbase + r * RC, RC)], gsem0)
    for r in range(NRC):
        pltpu.make_async_copy(acc_sh.at[pl.ds(rbase + r * RC, RC)],
                              out_h.at[c, pl.ds(rbase + r * RC, RC)],
                              gsem0).wait()


@functools.cache
def _make_spmm(D):
    mesh = plsc.VectorSubcoreMesh(core_axis_name="c", subcore_axis_name="s",
                                  num_cores=NC, num_subcores=NS)
    return pl.kernel(
        functools.partial(_spmm_body, D),
        out_type=jax.ShapeDtypeStruct((NC, NP, D), jnp.float32),
        mesh=mesh,
        scratch_types=[
            pltpu.VMEM_SHARED((NP, D), jnp.float32),
            pltpu.VMEM((NCHUNK, C), jnp.int32),
            pltpu.VMEM((NCHUNK, C), jnp.int32),
            pltpu.VMEM((NCHUNK, C), jnp.float32),
        ] + [pltpu.VMEM((C, D), jnp.float32)] * 10
          + [pltpu.SemaphoreType.DMA] * 10,
        compiler_params=pltpu.CompilerParams(needs_layout_passes=False,
                                             use_tc_tiling_on_sc=False),
    )


_BLK = 1000


def _mm_body(x_ref, w_ref, o_ref):
    o_ref[...] = jnp.dot(x_ref[...], w_ref[...],
                         preferred_element_type=jnp.float32)


def _matmul(x, W):
    M, K = x.shape
    Dout = W.shape[1]
    return pl.pallas_call(
        _mm_body,
        grid=(M // _BLK,),
        in_specs=[pl.BlockSpec((_BLK, K), lambda i: (i, 0)),
                  pl.BlockSpec((K, Dout), lambda i: (0, 0))],
        out_specs=pl.BlockSpec((_BLK, Dout), lambda i: (i, 0)),
        out_shape=jax.ShapeDtypeStruct((M, Dout), jnp.float32),
    )(x, W)


def _cmm_body(p_ref, b_ref, w_ref, act_ref, sup_ref):
    act = jnp.maximum(p_ref[0] + p_ref[1] + b_ref[...], 0.0)
    act_ref[...] = act
    sup_ref[...] = jnp.dot(act, w_ref[...], preferred_element_type=jnp.float32)


def _combine_mm(P, b, W):
    D = P.shape[2]
    Dout = W.shape[1]
    return pl.pallas_call(
        _cmm_body,
        grid=(N // _BLK,),
        in_specs=[pl.BlockSpec((2, _BLK, D), lambda i: (0, i, 0)),
                  pl.BlockSpec((1, D), lambda i: (0, 0)),
                  pl.BlockSpec((D, Dout), lambda i: (0, 0))],
        out_specs=[pl.BlockSpec((_BLK, D), lambda i: (i, 0)),
                   pl.BlockSpec((_BLK, Dout), lambda i: (i, 0))],
        out_shape=[jax.ShapeDtypeStruct((N, D), jnp.float32),
                   jax.ShapeDtypeStruct((N, Dout), jnp.float32)],
    )(P, b.reshape(1, D), W)


def _c2_body(pa_ref, pb_ref, b_ref, act_ref):
    act = jnp.concatenate([pa_ref[0] + pa_ref[1], pb_ref[0] + pb_ref[1]],
                          axis=1)
    act_ref[...] = jnp.maximum(act + b_ref[...], 0.0)


def _combine2(Pa, Pb, b):
    Dh = Pa.shape[2]
    D = 2 * Dh
    return pl.pallas_call(
        _c2_body,
        grid=(N // _BLK,),
        in_specs=[pl.BlockSpec((2, _BLK, Dh), lambda i: (0, i, 0)),
                  pl.BlockSpec((2, _BLK, Dh), lambda i: (0, i, 0)),
                  pl.BlockSpec((1, D), lambda i: (0, 0))],
        out_specs=pl.BlockSpec((_BLK, D), lambda i: (i, 0)),
        out_shape=jax.ShapeDtypeStruct((N, D), jnp.float32),
    )(Pa, Pb, b.reshape(1, D))


def _branch(feat, ew, ei, Ws, bs):
    pad = ((0, 0), (0, EPT_P - EPT))
    dst = jnp.pad(ei[0].astype(jnp.int32).reshape(NW, EPT),
                  pad).reshape(NW, NCHUNK, C)
    src = jnp.pad(ei[1].astype(jnp.int32).reshape(NW, EPT),
                  pad).reshape(NW, NCHUNK, C)
    ew = jnp.pad(ew.reshape(NW, EPT), pad).reshape(NW, NCHUNK, C)
    s0 = _matmul(feat, Ws[0])
    P = _make_spmm(Ws[0].shape[1])(dst, src, ew, s0)
    _, s1 = _combine_mm(P, bs[0], Ws[1])
    P = _make_spmm(Ws[1].shape[1])(dst, src, ew, s1)
    p0, s2 = _combine_mm(P, bs[1], Ws[2])
    P = _make_spmm(Ws[2].shape[1])(dst, src, ew, s2)
    _, s3 = _combine_mm(P, bs[2], Ws[3])
    Dh = Ws[3].shape[1] // 2
    Pa = _make_spmm(Dh)(dst, src, ew, s3[:, :Dh])
    Pb = _make_spmm(Dh)(dst, src, ew, s3[:, Dh:])
    h1 = _combine2(Pa, Pb, bs[3])
    return p0, h1


def kernel(feature1, edge_weight1, feature2, edge_weight2, params,
           edge_index1, edge_index2):
    p0, h1 = _branch(feature1, edge_weight1, edge_index1,
                     params["Ws1"], params["bs1"])
    p3, h4 = _branch(feature2, edge_weight2, edge_index2,
                     params["Ws2"], params["bs2"])
    return (p0, h1, p3, h4)
